# baseline (speedup 1.0000x reference)
import jax
import jax.numpy as jnp
from jax.experimental import pallas as pl

E = 8
D_IN = 6
H = 256
D_OUT = 4
MARGIN = 1.25


def _fused_kernel(x_ref, c_ref, w1_ref, b1_ref, w2_ref, b2_ref, w3_ref, b3_ref,
                  out_ref):
    xt = x_ref[...]
    c = c_ref[...]
    d2 = jnp.zeros((xt.shape[0], E), dtype=jnp.float32)
    for j in range(3):
        diff = xt[:, j:j + 1] - c[:, j][None, :]
        d2 = d2 + diff * diff
    d = jnp.sqrt(d2)
    inv = 1.0 / (d + 1e-8)
    dmin = jnp.min(d, axis=1, keepdims=True)
    inv = jnp.where(d > MARGIN * dmin, 0.0, inv)
    w = inv / jnp.sum(inv, axis=1, keepdims=True)

    acc = jnp.zeros((xt.shape[0], D_OUT), dtype=jnp.float32)
    for e in range(E):
        h = jnp.dot(xt, w1_ref[e], preferred_element_type=jnp.float32)
        h = jax.nn.relu(h + b1_ref[e][None, :])
        h = jnp.dot(h, w2_ref[e], preferred_element_type=jnp.float32)
        h = jax.nn.relu(h + b2_ref[e][None, :])
        o = jnp.dot(h, w3_ref[e], preferred_element_type=jnp.float32)
        o = o + b3_ref[e][None, :]
        acc = acc + o * w[:, e:e + 1]
    out_ref[...] = acc


@jax.jit
def kernel(x, centroids, W1, b1, W2, b2, W3, b3):
    n = x.shape[0]
    B = 8192
    grid = (n // B,)
    out = pl.pallas_call(
        _fused_kernel,
        grid=grid,
        in_specs=[
            pl.BlockSpec((B, D_IN), lambda i: (i, 0)),
            pl.BlockSpec((E, 3), lambda i: (0, 0)),
            pl.BlockSpec((E, D_IN, H), lambda i: (0, 0, 0)),
            pl.BlockSpec((E, H), lambda i: (0, 0)),
            pl.BlockSpec((E, H, H), lambda i: (0, 0, 0)),
            pl.BlockSpec((E, H), lambda i: (0, 0)),
            pl.BlockSpec((E, H, D_OUT), lambda i: (0, 0, 0)),
            pl.BlockSpec((E, D_OUT), lambda i: (0, 0)),
        ],
        out_specs=pl.BlockSpec((B, D_OUT), lambda i: (i, 0)),
        out_shape=jax.ShapeDtypeStruct((n, D_OUT), jnp.float32),
    )(x, centroids, W1, b1, W2, b2, W3, b3)
    return out


# speedup vs baseline: 1.4930x; 1.4930x over previous
import jax
import jax.numpy as jnp
from jax import lax
from jax.experimental import pallas as pl

E = 8
D_IN = 6
H = 256
D_OUT = 4
MARGIN = 1.25
CONTRACT00 = (((0,), (0,)), ((), ()))


def _fused_kernel(x_ref, c_ref, w1_ref, b1_ref, w2_ref, b2_ref, w3_ref, b3_ref,
                  out_ref):
    xt = x_ref[...]
    c = c_ref[...]
    nb = xt.shape[1]
    d2 = jnp.zeros((E, nb), dtype=jnp.float32)
    for j in range(3):
        diff = c[:, j:j + 1] - xt[j:j + 1, :]
        d2 = d2 + diff * diff
    d = jnp.sqrt(d2)
    inv = 1.0 / (d + 1e-8)
    dmin = jnp.min(d, axis=0, keepdims=True)
    inv = jnp.where(d > MARGIN * dmin, 0.0, inv)
    w = inv / jnp.sum(inv, axis=0, keepdims=True)

    acc = jnp.zeros((D_OUT, nb), dtype=jnp.float32)
    for e in range(E):
        h = lax.dot_general(w1_ref[e], xt, CONTRACT00,
                            preferred_element_type=jnp.float32)
        h = jax.nn.relu(h + b1_ref[e][:, None])
        h = lax.dot_general(w2_ref[e], h, CONTRACT00,
                            preferred_element_type=jnp.float32)
        h = jax.nn.relu(h + b2_ref[e][:, None])
        o = lax.dot_general(w3_ref[e], h, CONTRACT00,
                            preferred_element_type=jnp.float32)
        o = o + b3_ref[e][:, None]
        acc = acc + o * w[e:e + 1, :]
    out_ref[...] = acc


@jax.jit
def kernel(x, centroids, W1, b1, W2, b2, W3, b3):
    n = x.shape[0]
    B = 4096
    xT = x.T
    grid = (n // B,)
    outT = pl.pallas_call(
        _fused_kernel,
        grid=grid,
        in_specs=[
            pl.BlockSpec((D_IN, B), lambda i: (0, i)),
            pl.BlockSpec((E, 3), lambda i: (0, 0)),
            pl.BlockSpec((E, D_IN, H), lambda i: (0, 0, 0)),
            pl.BlockSpec((E, H), lambda i: (0, 0)),
            pl.BlockSpec((E, H, H), lambda i: (0, 0, 0)),
            pl.BlockSpec((E, H), lambda i: (0, 0)),
            pl.BlockSpec((E, H, D_OUT), lambda i: (0, 0, 0)),
            pl.BlockSpec((E, D_OUT), lambda i: (0, 0)),
        ],
        out_specs=pl.BlockSpec((D_OUT, B), lambda i: (0, i)),
        out_shape=jax.ShapeDtypeStruct((D_OUT, n), jnp.float32),
    )(xT, centroids, W1, b1, W2, b2, W3, b3)
    return outT.T


# speedup vs baseline: 1.5526x; 1.0400x over previous
import jax
import jax.numpy as jnp
from jax import lax
from jax.experimental import pallas as pl

E = 8
D_IN = 6
H = 256
D_OUT = 4
MARGIN = 1.25
CONTRACT00 = (((0,), (0,)), ((), ()))


def _fused_kernel(x_ref, c_ref, w1_ref, b1_ref, w2_ref, b2_ref, w3_ref, b3_ref,
                  out_ref):
    xt = x_ref[...]
    c = c_ref[...]
    nb = xt.shape[1]
    d2 = jnp.zeros((E, nb), dtype=jnp.float32)
    for j in range(3):
        diff = c[:, j:j + 1] - xt[j:j + 1, :]
        d2 = d2 + diff * diff
    d = jnp.sqrt(d2)
    inv = 1.0 / (d + 1e-8)
    dmin = jnp.min(d, axis=0, keepdims=True)
    inv = jnp.where(d > MARGIN * dmin, 0.0, inv)
    w = inv / jnp.sum(inv, axis=0, keepdims=True)

    acc = jnp.zeros((D_OUT, nb), dtype=jnp.float32)
    for e in range(E):
        h = lax.dot_general(w1_ref[e], xt, CONTRACT00,
                            preferred_element_type=jnp.float32)
        h = jax.nn.relu(h + b1_ref[e][:, None])
        h = lax.dot_general(w2_ref[e], h, CONTRACT00,
                            preferred_element_type=jnp.float32)
        h = jax.nn.relu(h + b2_ref[e][:, None])
        o = lax.dot_general(w3_ref[e], h, CONTRACT00,
                            preferred_element_type=jnp.float32)
        o = o + b3_ref[e][:, None]
        acc = acc + o * w[e:e + 1, :]
    out_ref[...] = acc


@jax.jit
def kernel(x, centroids, W1, b1, W2, b2, W3, b3):
    n = x.shape[0]
    B = 8192
    xT = x.T
    grid = (n // B,)
    outT = pl.pallas_call(
        _fused_kernel,
        grid=grid,
        in_specs=[
            pl.BlockSpec((D_IN, B), lambda i: (0, i)),
            pl.BlockSpec((E, 3), lambda i: (0, 0)),
            pl.BlockSpec((E, D_IN, H), lambda i: (0, 0, 0)),
            pl.BlockSpec((E, H), lambda i: (0, 0)),
            pl.BlockSpec((E, H, H), lambda i: (0, 0, 0)),
            pl.BlockSpec((E, H), lambda i: (0, 0)),
            pl.BlockSpec((E, H, D_OUT), lambda i: (0, 0, 0)),
            pl.BlockSpec((E, D_OUT), lambda i: (0, 0)),
        ],
        out_specs=pl.BlockSpec((D_OUT, B), lambda i: (0, i)),
        out_shape=jax.ShapeDtypeStruct((D_OUT, n), jnp.float32),
    )(xT, centroids, W1, b1, W2, b2, W3, b3)
    return outT.T
